# baseline (speedup 1.0000x reference)
import functools

import jax
import jax.numpy as jnp
from jax import lax
from jax.experimental import pallas as pl
from jax.experimental.pallas import tpu as pltpu
from jax.experimental.pallas import tpu_sc as plsc

VOCAB = 100000
EMBED_DIM = 64
BATCH = 4096
SEQ = 200

_NC = 2
_NS = 16
_BPW = BATCH // _NS
_HALF = SEQ // 2
_LANES = 16
_FEAT = EMBED_DIM // _NC
_VPT = VOCAB // _NS
_IDEPTH = 8
_GDEPTH = 2
_ILEAD = 6
_GLEAD = 1
_NLOC_B = 24
_NLOC = _HALF
_NREM = _HALF


def _body(x_hbm, tbl_hbm, ftbl_hbm, out_hbm, shared, idx_v, lrows, rrows,
          out_stage, *sems):
    c = lax.axis_index("c")
    s = lax.axis_index("s")
    base = s * _BPW
    isems = sems[:_IDEPTH]
    lsems = sems[_IDEPTH:_IDEPTH + _GDEPTH]
    rsems = sems[_IDEPTH + _GDEPTH:]

    pltpu.sync_copy(tbl_hbm.at[c, pl.ds(s * _VPT, _VPT)],
                    shared.at[pl.ds(s * _VPT, _VPT)])
    plsc.subcore_barrier()

    def start_idx(b, k):
        pltpu.async_copy(x_hbm.at[base + b], idx_v.at[k], isems[k])

    def wait_idx(b, k):
        pltpu.make_async_copy(
            x_hbm.at[base + b], idx_v.at[k], isems[k]).wait()

    def start_gather(k, g):
        pltpu.async_copy(
            shared.at[idx_v.at[k, 0]], lrows.at[g], lsems[g])
        pltpu.async_copy(
            ftbl_hbm.at[idx_v.at[k, 1]], rrows.at[g], rsems[g])

    def wait_gather(k, g):
        pltpu.make_async_copy(
            shared.at[idx_v.at[k, 0]], lrows.at[g], lsems[g]).wait()
        pltpu.make_async_copy(
            ftbl_hbm.at[idx_v.at[k, 1]], rrows.at[g], rsems[g]).wait()

    for p in range(_ILEAD):
        start_idx(p, p % _IDEPTH)
    for p in range(_GLEAD):
        wait_idx(p, p % _IDEPTH)
        start_gather(p % _IDEPTH, p % _GDEPTH)

    def outer(i, carry):
        for k in range(_IDEPTH):
            b = _IDEPTH * i + k

            @pl.when(b + _ILEAD < _BPW)
            def _():
                start_idx(b + _ILEAD, (k + _ILEAD) % _IDEPTH)

            @pl.when(b + _GLEAD < _BPW)
            def _():
                wait_idx(b + _GLEAD, (k + _GLEAD) % _IDEPTH)
                start_gather((k + _GLEAD) % _IDEPTH, (k + _GLEAD) % _GDEPTH)

            wait_gather(k, k % _GDEPTH)

            v = lrows[k % _GDEPTH, 0, pl.ds(0, 2 * _LANES)]
            acc = plsc.unpack(v, format=plsc.PackFormat.INTERLEAVED)
            w2 = rrows[k % _GDEPTH, 0, pl.ds(0, 2 * _LANES)]
            w, _wb = plsc.unpack(w2, format=plsc.PackFormat.INTERLEAVED)
            scale = jnp.float32(1.0 / SEQ)
            lanes = lax.iota(jnp.int32, 16)
            plsc.store_scatter(out_stage.at[b], [lanes * 2],
                               (acc[0] + w) * scale)
            plsc.store_scatter(out_stage.at[b], [lanes * 2 + 1],
                               acc[1] * scale)
        return carry

    lax.fori_loop(0, _BPW // _IDEPTH, outer, 0)
    pltpu.sync_copy(out_stage,
                    out_hbm.at[pl.ds(base, _BPW), pl.ds(c * _FEAT, _FEAT)])


def kernel(x, table):
    x3 = x.reshape(BATCH, 2, _HALF)
    tbl = table.astype(jnp.bfloat16).reshape(VOCAB, _NC, _FEAT)
    tbl = tbl.transpose(1, 0, 2)
    mesh = plsc.VectorSubcoreMesh(core_axis_name="c", subcore_axis_name="s")
    f = functools.partial(
        pl.kernel,
        out_type=jax.ShapeDtypeStruct((BATCH, EMBED_DIM), jnp.float32),
        mesh=mesh,
        scratch_types=[
            pltpu.VMEM_SHARED((VOCAB, _FEAT), jnp.bfloat16),
            pltpu.VMEM((_IDEPTH, 2, _HALF), jnp.int32),
            pltpu.VMEM((_GDEPTH, _NLOC, _FEAT), jnp.bfloat16),
            pltpu.VMEM((_GDEPTH, _NREM, EMBED_DIM), jnp.bfloat16),
            pltpu.VMEM((_BPW, _FEAT), jnp.float32),
        ] + [pltpu.SemaphoreType.DMA] * (_IDEPTH + 2 * _GDEPTH),
        compiler_params=pltpu.CompilerParams(
            use_tc_tiling_on_sc=False, needs_layout_passes=False),
    )(_body)
    return f(x3, tbl, table.astype(jnp.bfloat16))


# speedup vs baseline: 1.8231x; 1.8231x over previous
import functools

import jax
import jax.numpy as jnp
from jax import lax
from jax.experimental import pallas as pl
from jax.experimental.pallas import tpu as pltpu
from jax.experimental.pallas import tpu_sc as plsc

VOCAB = 100000
EMBED_DIM = 64
BATCH = 4096
SEQ = 200

_NC = 2
_NS = 16
_NW = _NC * _NS
_BPW = BATCH // _NW
_HALF = SEQ // 2
_LANES = 16
_DREG = EMBED_DIM // _LANES


_NBUF = 8


def _body(x_hbm, table_hbm, out_hbm, idx_all, rows_v, out_stage, *sems):
    wid = lax.axis_index("s") * _NC + lax.axis_index("c")
    base = wid * _BPW

    pltpu.sync_copy(x_hbm.at[pl.ds(base, _BPW)], idx_all)

    def start(slot, b):
        pltpu.async_copy(
            table_hbm.at[idx_all.at[b, 0]],
            rows_v.at[slot, pl.ds(0, _HALF)], sems[slot])
        pltpu.async_copy(
            table_hbm.at[idx_all.at[b, 1]],
            rows_v.at[slot, pl.ds(_HALF, _HALF)], sems[slot])

    def wait(slot, b):
        pltpu.make_async_copy(
            table_hbm.at[idx_all.at[b, 0]],
            rows_v.at[slot, pl.ds(0, _HALF)], sems[slot]).wait()
        pltpu.make_async_copy(
            table_hbm.at[idx_all.at[b, 1]],
            rows_v.at[slot, pl.ds(_HALF, _HALF)], sems[slot]).wait()

    for p in range(_NBUF - 1):
        start(p, p)

    def outer(i, carry):
        for k in range(_NBUF):
            b = _NBUF * i + k
            nxt = b + _NBUF - 1

            @pl.when(nxt < _BPW)
            def _():
                start((k + _NBUF - 1) % _NBUF, nxt)

            wait(k, b)

            def accum(s, acc):
                out = []
                for h in range(2):
                    v = rows_v[k, s, pl.ds(h * 2 * _LANES, 2 * _LANES)]
                    pa, pb = plsc.unpack(
                        v, format=plsc.PackFormat.INTERLEAVED)
                    out.append(acc[2 * h] + pa)
                    out.append(acc[2 * h + 1] + pb)
                return tuple(out)

            zero = jnp.zeros((_LANES,), jnp.float32)
            acc = lax.fori_loop(0, SEQ, accum, (zero,) * 4, unroll=8)
            scale = jnp.float32(1.0 / SEQ)
            lanes = lax.iota(jnp.int32, 16)
            for h in range(2):
                idx_a = lanes * 2 + (h * 2 * _LANES)
                plsc.store_scatter(
                    out_stage.at[b], [idx_a], acc[2 * h] * scale)
                plsc.store_scatter(
                    out_stage.at[b], [idx_a + 1], acc[2 * h + 1] * scale)
        return carry

    lax.fori_loop(0, _BPW // _NBUF, outer, 0)
    pltpu.sync_copy(out_stage, out_hbm.at[pl.ds(base, _BPW)])


def kernel(x, table):
    x3 = x.reshape(BATCH, 2, _HALF)
    table = table.astype(jnp.bfloat16)
    mesh = plsc.VectorSubcoreMesh(core_axis_name="c", subcore_axis_name="s")
    f = functools.partial(
        pl.kernel,
        out_type=jax.ShapeDtypeStruct((BATCH, EMBED_DIM), jnp.float32),
        mesh=mesh,
        scratch_types=[
            pltpu.VMEM((_BPW, 2, _HALF), jnp.int32),
            pltpu.VMEM((_NBUF, SEQ, EMBED_DIM), jnp.bfloat16),
            pltpu.VMEM((_BPW, EMBED_DIM), jnp.float32),
        ] + [pltpu.SemaphoreType.DMA] * _NBUF,
        compiler_params=pltpu.CompilerParams(
            use_tc_tiling_on_sc=False, needs_layout_passes=False),
    )(_body)
    return f(x3, table)
